# Initial kernel scaffold; baseline (speedup 1.0000x reference)
import jax
import jax.numpy as jnp
from jax.experimental import pallas as pl


def kernel(xyz, new_xyz, features, W1, b1, W2, b2, W3, b3):
    raise NotImplementedError("write your pallas kernel here")


# speedup vs baseline: 28.2400x; 28.2400x over previous
import functools

import jax
import jax.numpy as jnp
from jax import lax
from jax.experimental import pallas as pl
from jax.experimental.pallas import tpu as pltpu
from jax.experimental.pallas import tpu_sc as plsc

B, N, M, C = 4, 8192, 2048, 64
NS = 32
R2 = 0.4 * 0.4
BM = B * M
NW = 32
CPW = BM // NW
WPB = M // CPW
KCH = (NS * BM) // NW // 128


def _sc_ballquery(xyz_t, new_t):
    mesh = plsc.VectorSubcoreMesh(core_axis_name="c", subcore_axis_name="s")

    @functools.partial(
        pl.kernel,
        out_type=(
            jax.ShapeDtypeStruct((NW, NS * CPW), jnp.int32),
            jax.ShapeDtypeStruct((NW, CPW * NS), jnp.float32),
            jax.ShapeDtypeStruct((NW, CPW * NS), jnp.float32),
            jax.ShapeDtypeStruct((NW, CPW * NS), jnp.float32),
        ),
        mesh=mesh,
        compiler_params=pltpu.CompilerParams(needs_layout_passes=False),
        scratch_types=[
            pltpu.VMEM((N,), jnp.float32),
            pltpu.VMEM((N,), jnp.float32),
            pltpu.VMEM((N,), jnp.float32),
            pltpu.VMEM((CPW,), jnp.float32),
            pltpu.VMEM((CPW,), jnp.float32),
            pltpu.VMEM((CPW,), jnp.float32),
            pltpu.VMEM((NS * CPW,), jnp.int32),
            pltpu.VMEM((CPW * NS,), jnp.float32),
            pltpu.VMEM((CPW * NS,), jnp.float32),
            pltpu.VMEM((CPW * NS,), jnp.float32),
        ],
    )
    def k(xyz_hbm, new_hbm, idx_hbm, gxx_hbm, gxy_hbm, gxz_hbm,
          px_v, py_v, pz_v, cx_v, cy_v, cz_v, idx_v, gxx_v, gxy_v, gxz_v):
        wid = lax.axis_index("c") * 16 + lax.axis_index("s")
        b = wid // WPB
        moff = (wid % WPB) * CPW
        pltpu.sync_copy(xyz_hbm.at[pl.ds((b * 3 + 0) * N, N)], px_v)
        pltpu.sync_copy(xyz_hbm.at[pl.ds((b * 3 + 1) * N, N)], py_v)
        pltpu.sync_copy(xyz_hbm.at[pl.ds((b * 3 + 2) * N, N)], pz_v)
        pltpu.sync_copy(new_hbm.at[pl.ds((b * 3 + 0) * M + moff, CPW)], cx_v)
        pltpu.sync_copy(new_hbm.at[pl.ds((b * 3 + 1) * M + moff, CPW)], cy_v)
        pltpu.sync_copy(new_hbm.at[pl.ds((b * 3 + 2) * M + moff, CPW)], cz_v)

        lanes = lax.iota(jnp.int32, 16)
        zeros16 = jnp.zeros((16,), jnp.int32)
        ones16 = jnp.full((16,), 1, jnp.int32)
        twos16 = jnp.full((16,), 2, jnp.int32)
        boff = b * N

        def center_body(m, carry):
            mvec = jnp.full((16,), m, jnp.int32)
            mq = (m // 16) * 16
            ml = jnp.full((16,), lax.rem(m, 16), jnp.int32)
            cx = cx_v[pl.ds(mq, 16)].at[ml].get(mode="promise_in_bounds")
            cy = cy_v[pl.ds(mq, 16)].at[ml].get(mode="promise_in_bounds")
            cz = cz_v[pl.ds(mq, 16)].at[ml].get(mode="promise_in_bounds")

            fgi0 = jnp.full((16,), boff, jnp.int32)
            p0 = px_v[pl.ds(0, 16)]
            q0 = py_v[pl.ds(0, 16)]
            r0 = pz_v[pl.ds(0, 16)]
            fx0 = p0.at[zeros16].get(mode="promise_in_bounds") - cx
            fy0 = q0.at[zeros16].get(mode="promise_in_bounds") - cy
            fz0 = r0.at[zeros16].get(mode="promise_in_bounds") - cz

            def cond(st):
                return (st[0] < N // 16) & (st[1] < NS)

            def body(st):
                j, cnt, fgi, fx, fy, fz = st
                base = j * 16
                px = px_v[pl.ds(base, 16)]
                py = py_v[pl.ds(base, 16)]
                pz = pz_v[pl.ds(base, 16)]
                dx = px - cx
                dy = py - cy
                dz = pz - cz
                sq = dx * dx + dy * dy + dz * dz
                msk = sq < R2
                mi = msk.astype(jnp.int32)
                pre = plsc.cumsum(mi)
                pos = cnt + pre - 1
                okm = msk & (pos < NS)
                jv = base + lanes + boff
                plsc.store_scatter(idx_v, [pos * CPW + mvec], jv, mask=okm)
                plsc.store_scatter(gxx_v, [mvec * NS + pos], dx, mask=okm)
                plsc.store_scatter(gxy_v, [mvec * NS + pos], dy, mask=okm)
                plsc.store_scatter(gxz_v, [mvec * NS + pos], dz, mask=okm)
                nm = jnp.sum(mi)
                is_first = (cnt == 0) & (nm > 0)
                ffs = plsc.all_reduce_ffs(msk)
                ffc = jnp.minimum(ffs, 15)
                fgi = jnp.where(is_first, base + ffs + boff, fgi)
                fx = jnp.where(is_first,
                               dx.at[ffc].get(mode="promise_in_bounds"), fx)
                fy = jnp.where(is_first,
                               dy.at[ffc].get(mode="promise_in_bounds"), fy)
                fz = jnp.where(is_first,
                               dz.at[ffc].get(mode="promise_in_bounds"), fz)
                return j + 1, cnt + nm, fgi, fx, fy, fz

            _, cnt, fgiv, fxv, fyv, fzv = lax.while_loop(
                cond, body,
                (jnp.int32(0), jnp.int32(0), fgi0, fx0, fy0, fz0))

            @pl.when(cnt < NS)
            def _fill():
                for h in (0, 16):
                    lp = lanes + h
                    fm = lp >= cnt
                    plsc.store_scatter(idx_v, [lp * CPW + mvec], fgiv,
                                       mask=fm)
                    plsc.store_scatter(gxx_v, [mvec * NS + lp], fxv, mask=fm)
                    plsc.store_scatter(gxy_v, [mvec * NS + lp], fyv, mask=fm)
                    plsc.store_scatter(gxz_v, [mvec * NS + lp], fzv, mask=fm)

            return carry

        lax.fori_loop(0, CPW, center_body, 0)
        pltpu.sync_copy(idx_v, idx_hbm.at[wid])
        pltpu.sync_copy(gxx_v, gxx_hbm.at[wid])
        pltpu.sync_copy(gxy_v, gxy_hbm.at[wid])
        pltpu.sync_copy(gxz_v, gxz_hbm.at[wid])

    return k(xyz_t, new_t)


CP = 128


def _sc_gather(feat_rows, idx2d):
    mesh = plsc.VectorSubcoreMesh(core_axis_name="c", subcore_axis_name="s")

    @functools.partial(
        pl.kernel,
        out_type=jax.ShapeDtypeStruct((NS * BM, CP), jnp.float32),
        mesh=mesh,
        compiler_params=pltpu.CompilerParams(needs_layout_passes=False),
        scratch_types=[
            pltpu.VMEM((KCH, 128), jnp.int32),
            pltpu.VMEM((128, CP), jnp.float32),
            pltpu.VMEM((128, CP), jnp.float32),
            pltpu.SemaphoreType.DMA,
            pltpu.SemaphoreType.DMA,
        ],
    )
    def k(feat_hbm, idx_hbm, out_hbm, idx_v, rows_a, rows_b, sem_a, sem_b):
        wid = lax.axis_index("c") * 16 + lax.axis_index("s")
        pltpu.sync_copy(idx_hbm.at[pl.ds(wid * KCH, KCH), :], idx_v)
        rbase = wid * KCH * 128

        cp0 = pltpu.async_copy(feat_hbm.at[idx_v.at[0]], rows_a, sem_a)

        def body(j, _):
            even = lax.rem(j, 2) == 0

            @pl.when((j + 1) < KCH)
            def _prefetch():
                @pl.when(even)
                def _():
                    pltpu.async_copy(feat_hbm.at[idx_v.at[j + 1]], rows_b,
                                     sem_b)

                @pl.when(jnp.logical_not(even))
                def _():
                    pltpu.async_copy(feat_hbm.at[idx_v.at[j + 1]], rows_a,
                                     sem_a)

            @pl.when(even)
            def _drain_a():
                pltpu.make_async_copy(feat_hbm.at[idx_v.at[j]], rows_a,
                                      sem_a).wait()
                pltpu.sync_copy(rows_a, out_hbm.at[pl.ds(rbase + j * 128, 128), :])

            @pl.when(jnp.logical_not(even))
            def _drain_b():
                pltpu.make_async_copy(feat_hbm.at[idx_v.at[j]], rows_b,
                                      sem_b).wait()
                pltpu.sync_copy(rows_b, out_hbm.at[pl.ds(rbase + j * 128, 128), :])

            return 0

        lax.fori_loop(0, KCH, body, 0)

    return k(feat_rows, idx2d)


def _tc_mlp(gf3, gxx, gxy, gxz, W1, b1, W2, b2, W3, b3):
    TM = 512
    W1xT = W1[:, :3].T
    W1fT = W1[:, 3:].T
    W2T = W2.T
    W3T = W3.T
    b1r = b1.reshape(1, 64)
    b2r = b2.reshape(1, 64)
    b3r = b3.reshape(1, 128)

    def body(gf_ref, gxx_ref, gxy_ref, gxz_ref, w1x_ref, w1f_ref, w2_ref,
             w3_ref, b1_ref, b2_ref, b3_ref, out_ref):
        gx = gxx_ref[...]
        gy = gxy_ref[...]
        gz = gxz_ref[...]
        sq = gx * gx + gy * gy + gz * gz
        dr = 1.0 / (jnp.sqrt(sq) + 1e-8)
        w = dr / jnp.sum(dr, axis=1, keepdims=True)
        w1x = w1x_ref[...]
        w1f = w1f_ref[...]
        w2 = w2_ref[...]
        w3 = w3_ref[...]
        bb1 = b1_ref[...]
        bb2 = b2_ref[...]
        bb3 = b3_ref[...]
        acc = jnp.zeros((TM, 128), jnp.float32)
        for ns in range(NS):
            xf = gf_ref[ns]
            xc = (gx[:, ns:ns + 1] * w1x[0:1, :]
                  + gy[:, ns:ns + 1] * w1x[1:2, :]
                  + gz[:, ns:ns + 1] * w1x[2:3, :])
            h = jnp.maximum(
                jnp.dot(xf, w1f, preferred_element_type=jnp.float32,
                        precision=lax.Precision.HIGHEST) + xc + bb1, 0.0)
            h = jnp.maximum(
                jnp.dot(h, w2, preferred_element_type=jnp.float32,
                        precision=lax.Precision.HIGHEST) + bb2, 0.0)
            h = jnp.maximum(
                jnp.dot(h, w3, preferred_element_type=jnp.float32,
                        precision=lax.Precision.HIGHEST) + bb3, 0.0)
            acc = acc + w[:, ns:ns + 1] * h
        out_ref[...] = acc

    grid = (BM // TM,)
    return pl.pallas_call(
        body,
        grid=grid,
        compiler_params=pltpu.CompilerParams(
            vmem_limit_bytes=100 * 1024 * 1024),
        in_specs=[
            pl.BlockSpec((NS, TM, C), lambda i: (0, i, 0)),
            pl.BlockSpec((TM, NS), lambda i: (i, 0)),
            pl.BlockSpec((TM, NS), lambda i: (i, 0)),
            pl.BlockSpec((TM, NS), lambda i: (i, 0)),
            pl.BlockSpec((3, 64), lambda i: (0, 0)),
            pl.BlockSpec((64, 64), lambda i: (0, 0)),
            pl.BlockSpec((64, 64), lambda i: (0, 0)),
            pl.BlockSpec((64, 128), lambda i: (0, 0)),
            pl.BlockSpec((1, 64), lambda i: (0, 0)),
            pl.BlockSpec((1, 64), lambda i: (0, 0)),
            pl.BlockSpec((1, 128), lambda i: (0, 0)),
        ],
        out_specs=pl.BlockSpec((TM, 128), lambda i: (i, 0)),
        out_shape=jax.ShapeDtypeStruct((BM, 128), jnp.float32),
    )(gf3, gxx, gxy, gxz, W1xT, W1fT, W2T, W3T, b1r, b2r, b3r)


def kernel(xyz, new_xyz, features, W1, b1, W2, b2, W3, b3):
    xyz_t = jnp.transpose(xyz, (0, 2, 1)).reshape(-1)
    new_t = jnp.transpose(new_xyz, (0, 2, 1)).reshape(-1)
    idx_raw, gxx_raw, gxy_raw, gxz_raw = _sc_ballquery(xyz_t, new_t)
    idx = idx_raw.reshape(NW, NS, CPW).transpose(1, 0, 2).reshape(NS, BM)
    gxx = gxx_raw.reshape(BM, NS)
    gxy = gxy_raw.reshape(BM, NS)
    gxz = gxz_raw.reshape(BM, NS)
    feat_rows = jnp.transpose(features, (0, 2, 1)).reshape(B * N, C)
    feat_rows = jnp.pad(feat_rows, ((0, 0), (0, CP - C)))
    gf = _sc_gather(feat_rows, idx.reshape(-1, 128))
    out = _tc_mlp(gf.reshape(NS, BM, CP)[:, :, :C], gxx, gxy, gxz,
                  W1, b1, W2, b2, W3, b3)
    out = out.reshape(B, M, 128).transpose(0, 2, 1)
    return (new_xyz, out)
